# baseline (speedup 1.0000x reference)
import functools

import jax
import jax.numpy as jnp
from jax import lax
from jax.experimental import pallas as pl
from jax.experimental.pallas import tpu as pltpu
from jax.experimental.pallas import tpu_sc as plsc

_NC, _NS = 2, 16
_NW = _NC * _NS
_CHUNK = 120
_NBUF = 2


@functools.lru_cache(maxsize=None)
def _make_sc_gather(V, D, B):
    b_per_w = B // _NW
    nchunks = b_per_w // _CHUNK
    mesh = plsc.VectorSubcoreMesh(core_axis_name="c", subcore_axis_name="s")

    @functools.partial(
        pl.kernel,
        out_type=jax.ShapeDtypeStruct((B, D), jnp.float32),
        mesh=mesh,
        scratch_types=[
            pltpu.VMEM((b_per_w,), jnp.int32),
            pltpu.VMEM((_NBUF, _CHUNK, D), jnp.float32),
            [pltpu.SemaphoreType.DMA] * _NBUF,
        ],
    )
    def sc_gather(table_hbm, idx_hbm, out_hbm, idx_v, buf_v, sems):
        wid = lax.axis_index("s") * _NC + lax.axis_index("c")
        base = wid * b_per_w
        pltpu.sync_copy(idx_hbm.at[pl.ds(base, b_per_w)], idx_v)

        def start(ci, b):
            pltpu.async_copy(
                table_hbm.at[idx_v.at[pl.ds(ci * _CHUNK, _CHUNK)]],
                buf_v.at[b], sems[b])

        def wait(b):
            pltpu.make_async_copy(
                table_hbm.at[idx_v.at[pl.ds(0, _CHUNK)]],
                buf_v.at[b], sems[b]).wait()

        for b in range(_NBUF):
            start(b, b)

        def body(j, carry):
            for b in range(_NBUF):
                ci = j * _NBUF + b
                wait(b)
                pltpu.sync_copy(
                    buf_v.at[b],
                    out_hbm.at[pl.ds(base + ci * _CHUNK, _CHUNK)])

                @pl.when(ci + _NBUF < nchunks)
                def _():
                    start(ci + _NBUF, b)
            return carry

        lax.fori_loop(0, nchunks // _NBUF, body, 0)

    return sc_gather


def _attention_body(e_ref, u_ref, w1_ref, b1_ref, w2_ref, b2_ref, w3_ref,
                    o_ref):
    bn, d = u_ref.shape
    k = e_ref.shape[0] // bn
    e3 = e_ref[...].reshape(k, bn, d)
    e2 = e_ref[...].astype(jnp.bfloat16)
    w1 = w1_ref[...]
    pn = jnp.dot(u_ref[...], w1[d:, :],
                 preferred_element_type=jnp.float32) + b1_ref[...]
    h = jnp.dot(e2, w1[:d, :].astype(jnp.bfloat16),
                preferred_element_type=jnp.float32)
    h = h + jnp.broadcast_to(pn[None, :, :], (k, bn, d)).reshape(k * bn, d)
    h = jnp.maximum(h, 0.0).astype(jnp.bfloat16)
    h = jnp.dot(h, w2_ref[...].astype(jnp.bfloat16),
                preferred_element_type=jnp.float32)
    h = jnp.maximum(h + b2_ref[...], 0.0).astype(jnp.bfloat16)
    s = jnp.dot(h, w3_ref[...].astype(jnp.bfloat16),
                preferred_element_type=jnp.float32)
    w = jnp.exp(s).reshape(k, bn, 1)
    wb = jnp.broadcast_to(w, (k, bn, d))
    num = jnp.sum(wb * e3, axis=0)
    den = jnp.sum(wb, axis=0)
    o_ref[...] = num / den


@functools.lru_cache(maxsize=None)
def _make_attention(N, K, D, BN, B):
    grid = (N // BN,)
    ublk0 = N * K // BN
    return pl.pallas_call(
        _attention_body,
        grid=grid,
        in_specs=[
            pl.BlockSpec((BN * K, D), lambda i: (i, 0)),
            pl.BlockSpec((BN, D), lambda i: (i + ublk0, 0)),
            pl.BlockSpec((2 * D, D), lambda i: (0, 0)),
            pl.BlockSpec((1, D), lambda i: (0, 0)),
            pl.BlockSpec((D, D), lambda i: (0, 0)),
            pl.BlockSpec((1, D), lambda i: (0, 0)),
            pl.BlockSpec((D, 1), lambda i: (0, 0)),
        ],
        out_specs=pl.BlockSpec((BN, D), lambda i: (i, 0)),
        out_shape=jax.ShapeDtypeStruct((N, D), jnp.float32),
        compiler_params=pltpu.CompilerParams(
            dimension_semantics=("parallel",)),
    )


def kernel(nodes, to_neighs, u2e, W1, b1, W2, b2, W3, b3):
    N, K = to_neighs.shape
    V, D = u2e.shape
    nidx = N * K + N
    unit = _NW * _CHUNK * _NBUF
    B = ((nidx + unit - 1) // unit) * unit
    bn = 400
    idx_e = to_neighs.reshape(N // bn, bn, K).transpose(0, 2, 1).reshape(-1)
    all_idx = jnp.concatenate([
        idx_e,
        nodes,
        jnp.zeros((B - nidx,), jnp.int32),
    ])
    gathered = _make_sc_gather(V, D, B)(u2e, all_idx)
    return _make_attention(N, K, D, bn, B)(
        gathered, gathered, W1, b1.reshape(1, D), W2, b2.reshape(1, D), W3)


# speedup vs baseline: 1.0049x; 1.0049x over previous
import functools

import jax
import jax.numpy as jnp
from jax import lax
from jax.experimental import pallas as pl
from jax.experimental.pallas import tpu as pltpu
from jax.experimental.pallas import tpu_sc as plsc

_NC, _NS = 2, 16
_NW = _NC * _NS
_CHUNK = 120
_NBUF = 2


@functools.lru_cache(maxsize=None)
def _make_sc_gather(V, D, B):
    b_per_w = B // _NW
    nchunks = b_per_w // _CHUNK
    mesh = plsc.VectorSubcoreMesh(core_axis_name="c", subcore_axis_name="s")

    @functools.partial(
        pl.kernel,
        out_type=jax.ShapeDtypeStruct((B, D), jnp.float32),
        mesh=mesh,
        scratch_types=[
            pltpu.VMEM((b_per_w,), jnp.int32),
            pltpu.VMEM((_NBUF, _CHUNK, D), jnp.float32),
            [pltpu.SemaphoreType.DMA] * _NBUF,
        ],
    )
    def sc_gather(table_hbm, idx_hbm, out_hbm, idx_v, buf_v, sems):
        wid = lax.axis_index("s") * _NC + lax.axis_index("c")
        base = wid * b_per_w
        pltpu.sync_copy(idx_hbm.at[pl.ds(base, b_per_w)], idx_v)

        def start(ci, b):
            pltpu.async_copy(
                table_hbm.at[idx_v.at[pl.ds(ci * _CHUNK, _CHUNK)]],
                buf_v.at[b], sems[b])

        def wait(b):
            pltpu.make_async_copy(
                table_hbm.at[idx_v.at[pl.ds(0, _CHUNK)]],
                buf_v.at[b], sems[b]).wait()

        for b in range(_NBUF):
            start(b, b)

        def body(j, carry):
            for b in range(_NBUF):
                ci = j * _NBUF + b
                wait(b)
                pltpu.sync_copy(
                    buf_v.at[b],
                    out_hbm.at[pl.ds(base + ci * _CHUNK, _CHUNK)])

                @pl.when(ci + _NBUF < nchunks)
                def _():
                    start(ci + _NBUF, b)
            return carry

        lax.fori_loop(0, nchunks // _NBUF, body, 0)

    return sc_gather


def _attention_body(e_ref, u_ref, w1_ref, b1_ref, w2_ref, b2_ref, w3_ref,
                    o_ref):
    bn, d = u_ref.shape
    k = e_ref.shape[0] // bn
    e3 = e_ref[...].reshape(k, bn, d)
    e2 = e_ref[...]
    w1 = w1_ref[...]
    pn = jnp.dot(u_ref[...], w1[d:, :],
                 preferred_element_type=jnp.float32) + b1_ref[...]
    h = jnp.dot(e2, w1[:d, :],
                preferred_element_type=jnp.float32)
    h = h + jnp.broadcast_to(pn[None, :, :], (k, bn, d)).reshape(k * bn, d)
    h = jnp.maximum(h, 0.0).astype(jnp.bfloat16)
    h = jnp.dot(h, w2_ref[...].astype(jnp.bfloat16),
                preferred_element_type=jnp.float32)
    h = jnp.maximum(h + b2_ref[...], 0.0).astype(jnp.bfloat16)
    s = jnp.dot(h, w3_ref[...].astype(jnp.bfloat16),
                preferred_element_type=jnp.float32)
    w = jnp.exp(s).reshape(k, bn, 1)
    wb = jnp.broadcast_to(w, (k, bn, d))
    num = jnp.sum(wb * e3, axis=0)
    den = jnp.sum(wb, axis=0)
    o_ref[...] = num / den


@functools.lru_cache(maxsize=None)
def _make_attention(N, K, D, BN, B):
    grid = (N // BN,)
    ublk0 = N * K // BN
    return pl.pallas_call(
        _attention_body,
        grid=grid,
        in_specs=[
            pl.BlockSpec((BN * K, D), lambda i: (i, 0)),
            pl.BlockSpec((BN, D), lambda i: (i + ublk0, 0)),
            pl.BlockSpec((2 * D, D), lambda i: (0, 0)),
            pl.BlockSpec((1, D), lambda i: (0, 0)),
            pl.BlockSpec((D, D), lambda i: (0, 0)),
            pl.BlockSpec((1, D), lambda i: (0, 0)),
            pl.BlockSpec((D, 1), lambda i: (0, 0)),
        ],
        out_specs=pl.BlockSpec((BN, D), lambda i: (i, 0)),
        out_shape=jax.ShapeDtypeStruct((N, D), jnp.float32),
        compiler_params=pltpu.CompilerParams(
            dimension_semantics=("parallel",)),
    )


def kernel(nodes, to_neighs, u2e, W1, b1, W2, b2, W3, b3):
    N, K = to_neighs.shape
    V, D = u2e.shape
    nidx = N * K + N
    unit = _NW * _CHUNK * _NBUF
    B = ((nidx + unit - 1) // unit) * unit
    bn = 400
    idx_e = to_neighs.reshape(N // bn, bn, K).transpose(0, 2, 1).reshape(-1)
    all_idx = jnp.concatenate([
        idx_e,
        nodes,
        jnp.zeros((B - nidx,), jnp.int32),
    ])
    gathered = _make_sc_gather(V, D, B)(u2e, all_idx)
    return _make_attention(N, K, D, bn, B)(
        gathered, gathered, W1, b1.reshape(1, D), W2, b2.reshape(1, D), W3)
